# Initial kernel scaffold; baseline (speedup 1.0000x reference)
import jax
import jax.numpy as jnp
from jax.experimental import pallas as pl


def kernel(omics_0, omics_1, omics_2, batch_size, params):
    raise NotImplementedError("write your pallas kernel here")


# speedup vs baseline: 96.6901x; 96.6901x over previous
import math

import jax
import jax.numpy as jnp
from jax.experimental import pallas as pl

HIDDEN = 64
HEADS = 4
NEG = -1e30


def _stats_kernel(x0_ref, x1_ref, x2_ref, o_ref):
    a, b, c = x0_ref[...], x1_ref[...], x2_ref[...]
    vals = (a, b, c, a * a, b * b, c * c, a * b, a * c, b * c)
    o_ref[...] = jnp.concatenate(
        [jnp.sum(v, keepdims=True) for v in vals], axis=1)


def _lrelu(x):
    return jnp.where(x >= 0, x, 0.2 * x)


def _fwd_kernel(x0_ref, x1_ref, x2_ref, ab_ref,
                u_ref, v_ref, s_ref, t_ref,
                w1_ref, a1s_ref, a1d_ref, b1_ref,
                w2_ref, a2s_ref, a2d_ref, b2_ref,
                cw1_ref, cb1_ref, cw2_ref, cb2_ref,
                out_ref, pooled_ref):
    xs = (x0_ref[...], x1_ref[...], x2_ref[...])

    n = []
    for i in range(3):
        h = xs[i] * u_ref[i:i + 1, :] + v_ref[i:i + 1, :]
        h = jnp.maximum(h, 0.0)
        n.append(h * s_ref[i:i + 1, :] + t_ref[i:i + 1, :])

    w1 = w1_ref[...]
    h1 = [jnp.dot(ni, w1, preferred_element_type=jnp.float32) for ni in n]
    as1 = [jnp.dot(ni, a1s_ref[...], preferred_element_type=jnp.float32)
           for ni in n]
    ad1 = [jnp.dot(ni, a1d_ref[...], preferred_element_type=jnp.float32)
           for ni in n]
    x1 = []
    for j in range(3):
        al = [_lrelu(as1[i] + ad1[j]) + ab_ref[i:i + 1, j:j + 1]
              for i in range(3)]
        m = jnp.maximum(jnp.maximum(al[0], al[1]), al[2])
        e = [jnp.exp(a - m) for a in al]
        inv = 1.0 / (e[0] + e[1] + e[2] + 1e-16)
        att = [ei * inv for ei in e]
        parts = []
        for hd in range(4):
            lo = hd * HIDDEN
            acc = att[0][:, hd:hd + 1] * h1[0][:, lo:lo + HIDDEN]
            acc = acc + att[1][:, hd:hd + 1] * h1[1][:, lo:lo + HIDDEN]
            acc = acc + att[2][:, hd:hd + 1] * h1[2][:, lo:lo + HIDDEN]
            parts.append(acc)
        o = jnp.concatenate(parts, axis=1) + b1_ref[...]
        x1.append(jnp.where(o > 0, o, jnp.exp(jnp.minimum(o, 0.0)) - 1.0))

    w2 = w2_ref[...]
    h2 = [jnp.dot(xj, w2, preferred_element_type=jnp.float32) for xj in x1]
    as2 = [jnp.dot(xj, a2s_ref[...], preferred_element_type=jnp.float32)
           for xj in x1]
    ad2 = [jnp.dot(xj, a2d_ref[...], preferred_element_type=jnp.float32)
           for xj in x1]
    pooled = jnp.zeros_like(h2[0])
    for j in range(3):
        al = [_lrelu(as2[i] + ad2[j]) + ab_ref[i:i + 1, j:j + 1]
              for i in range(3)]
        m = jnp.maximum(jnp.maximum(al[0], al[1]), al[2])
        e = [jnp.exp(a - m) for a in al]
        inv = 1.0 / (e[0] + e[1] + e[2] + 1e-16)
        o = (e[0] * h2[0] + e[1] * h2[1] + e[2] * h2[2]) * inv
        pooled = pooled + (o + b2_ref[...])
    pooled = pooled * (1.0 / 3.0)
    pooled_ref[...] = pooled

    hc = jnp.dot(pooled, cw1_ref[...],
                 preferred_element_type=jnp.float32) + cb1_ref[...]
    hc = jnp.maximum(hc, 0.0)
    logit = jnp.dot(hc, cw2_ref[...],
                    preferred_element_type=jnp.float32) + cb2_ref[...]
    out_ref[...] = jax.nn.sigmoid(logit)


def kernel(omics_0, omics_1, omics_2, batch_size, params):
    B = omics_0.shape[0]
    f32 = jnp.float32

    U, V, S, T = [], [], [], []
    inv_bn = 1.0 / math.sqrt(1.0 + 1e-5)
    for i in range(3):
        c = jnp.cos(params[f"rot_{i}"]) + jnp.sin(params[f"rot_{i}"])
        U.append(params[f"lin_w_{i}"][0] * c)
        V.append(params[f"lin_b_{i}"] * c)
        S.append(params[f"bn_g_{i}"] * inv_bn)
        T.append(params[f"bn_b_{i}"])
    U, V, S, T = (jnp.stack(a) for a in (U, V, S, T))

    w1 = params["gat1_w"]
    a1s = jnp.stack([w1[:, h * HIDDEN:(h + 1) * HIDDEN]
                     @ params["gat1_att_src"][h] for h in range(HEADS)],
                    axis=1)
    a1d = jnp.stack([w1[:, h * HIDDEN:(h + 1) * HIDDEN]
                     @ params["gat1_att_dst"][h] for h in range(HEADS)],
                    axis=1)
    b1 = params["gat1_bias"].reshape(1, HEADS * HIDDEN)
    w2 = params["gat2_w"]
    a2s = (w2 @ params["gat2_att_src"][0]).reshape(-1, 1)
    a2d = (w2 @ params["gat2_att_dst"][0]).reshape(-1, 1)
    b2 = params["gat2_bias"].reshape(1, HIDDEN)
    cw1 = params["cls_w1"]
    cb1 = params["cls_b1"].reshape(1, HIDDEN)
    cw2 = params["cls_w2"]
    cb2 = params["cls_b2"].reshape(1, 1)

    if B % 128 == 0:
        shp = (B // 128, 128)
    else:
        shp = (B, 1)
    stats = pl.pallas_call(
        _stats_kernel,
        out_shape=jax.ShapeDtypeStruct((1, 9), f32),
    )(omics_0.reshape(shp), omics_1.reshape(shp), omics_2.reshape(shp))

    s = stats[0]
    mu = s[0:3] / B
    pm = jnp.stack([jnp.stack([s[3], s[6], s[7]]),
                    jnp.stack([s[6], s[4], s[8]]),
                    jnp.stack([s[7], s[8], s[5]])])
    cov = pm - B * (mu[:, None] * mu[None, :])
    d = jnp.sqrt(jnp.diagonal(cov))
    corr = cov / (d[:, None] * d[None, :])
    allowed = (corr > 0.3) | jnp.eye(3, dtype=bool)
    abias = jnp.where(allowed, 0.0, NEG).astype(f32)

    G = 512
    while B % G != 0:
        G //= 2
    nb = B // G
    row_spec = lambda shape: pl.BlockSpec(shape, lambda i: (0, 0))
    in_specs = [
        pl.BlockSpec((G, 1), lambda i: (i, 0)),
        pl.BlockSpec((G, 1), lambda i: (i, 0)),
        pl.BlockSpec((G, 1), lambda i: (i, 0)),
        row_spec((3, 3)),
        row_spec((3, HIDDEN)), row_spec((3, HIDDEN)),
        row_spec((3, HIDDEN)), row_spec((3, HIDDEN)),
        row_spec((HIDDEN, HEADS * HIDDEN)),
        row_spec((HIDDEN, HEADS)), row_spec((HIDDEN, HEADS)),
        row_spec((1, HEADS * HIDDEN)),
        row_spec((HEADS * HIDDEN, HIDDEN)),
        row_spec((HEADS * HIDDEN, 1)), row_spec((HEADS * HIDDEN, 1)),
        row_spec((1, HIDDEN)),
        row_spec((HIDDEN, HIDDEN)), row_spec((1, HIDDEN)),
        row_spec((HIDDEN, 1)), row_spec((1, 1)),
    ]
    out, pooled = pl.pallas_call(
        _fwd_kernel,
        grid=(nb,),
        in_specs=in_specs,
        out_specs=[pl.BlockSpec((G, 1), lambda i: (i, 0)),
                   pl.BlockSpec((G, HIDDEN), lambda i: (i, 0))],
        out_shape=[jax.ShapeDtypeStruct((B, 1), f32),
                   jax.ShapeDtypeStruct((B, HIDDEN), f32)],
    )(omics_0, omics_1, omics_2, abias, U, V, S, T,
      w1, a1s, a1d, b1, w2, a2s, a2d, b2, cw1, cb1, cw2, cb2)
    return out, pooled


# speedup vs baseline: 103.7491x; 1.0730x over previous
import math

import jax
import jax.numpy as jnp
from jax.experimental import pallas as pl

HIDDEN = 64
HEADS = 4
NEG = -1e30


def _stats_kernel(x0_ref, x1_ref, x2_ref, o_ref):
    a, b, c = x0_ref[...], x1_ref[...], x2_ref[...]
    vals = (a, b, c, a * a, b * b, c * c, a * b, a * c, b * c)
    o_ref[...] = jnp.concatenate(
        [jnp.sum(v, keepdims=True) for v in vals], axis=1)


def _lrelu(x):
    return jnp.where(x >= 0, x, 0.2 * x)


def _fwd_kernel(x0_ref, x1_ref, x2_ref, ab_ref,
                u_ref, v_ref, s_ref, t_ref,
                w1_ref, a1s_ref, a1d_ref, b1_ref,
                w2_ref, a2s_ref, a2d_ref, b2_ref,
                cw1_ref, cb1_ref, cw2_ref, cb2_ref,
                out_ref, pooled_ref):
    xs = (x0_ref[...], x1_ref[...], x2_ref[...])

    n = []
    for i in range(3):
        h = xs[i] * u_ref[i:i + 1, :] + v_ref[i:i + 1, :]
        h = jnp.maximum(h, 0.0)
        n.append(h * s_ref[i:i + 1, :] + t_ref[i:i + 1, :])

    w1 = w1_ref[...]
    h1 = [jnp.dot(ni, w1, preferred_element_type=jnp.float32) for ni in n]
    as1 = [jnp.dot(ni, a1s_ref[...], preferred_element_type=jnp.float32)
           for ni in n]
    ad1 = [jnp.dot(ni, a1d_ref[...], preferred_element_type=jnp.float32)
           for ni in n]
    x1 = []
    for j in range(3):
        al = [_lrelu(as1[i] + ad1[j]) + ab_ref[i:i + 1, j:j + 1]
              for i in range(3)]
        m = jnp.maximum(jnp.maximum(al[0], al[1]), al[2])
        e = [jnp.exp(a - m) for a in al]
        inv = 1.0 / (e[0] + e[1] + e[2] + 1e-16)
        att = [ei * inv for ei in e]
        parts = []
        for hd in range(4):
            lo = hd * HIDDEN
            acc = att[0][:, hd:hd + 1] * h1[0][:, lo:lo + HIDDEN]
            acc = acc + att[1][:, hd:hd + 1] * h1[1][:, lo:lo + HIDDEN]
            acc = acc + att[2][:, hd:hd + 1] * h1[2][:, lo:lo + HIDDEN]
            parts.append(acc)
        o = jnp.concatenate(parts, axis=1) + b1_ref[...]
        x1.append(jnp.where(o > 0, o, jnp.exp(jnp.minimum(o, 0.0)) - 1.0))

    w2 = w2_ref[...]
    h2 = [jnp.dot(xj, w2, preferred_element_type=jnp.float32) for xj in x1]
    as2 = [jnp.dot(xj, a2s_ref[...], preferred_element_type=jnp.float32)
           for xj in x1]
    ad2 = [jnp.dot(xj, a2d_ref[...], preferred_element_type=jnp.float32)
           for xj in x1]
    pooled = jnp.zeros_like(h2[0])
    for j in range(3):
        al = [_lrelu(as2[i] + ad2[j]) + ab_ref[i:i + 1, j:j + 1]
              for i in range(3)]
        m = jnp.maximum(jnp.maximum(al[0], al[1]), al[2])
        e = [jnp.exp(a - m) for a in al]
        inv = 1.0 / (e[0] + e[1] + e[2] + 1e-16)
        o = (e[0] * h2[0] + e[1] * h2[1] + e[2] * h2[2]) * inv
        pooled = pooled + (o + b2_ref[...])
    pooled = pooled * (1.0 / 3.0)
    pooled_ref[...] = pooled

    hc = jnp.dot(pooled, cw1_ref[...],
                 preferred_element_type=jnp.float32) + cb1_ref[...]
    hc = jnp.maximum(hc, 0.0)
    logit = jnp.dot(hc, cw2_ref[...],
                    preferred_element_type=jnp.float32) + cb2_ref[...]
    out_ref[...] = jax.nn.sigmoid(logit)


def kernel(omics_0, omics_1, omics_2, batch_size, params):
    B = omics_0.shape[0]
    f32 = jnp.float32

    U, V, S, T = [], [], [], []
    inv_bn = 1.0 / math.sqrt(1.0 + 1e-5)
    for i in range(3):
        c = jnp.cos(params[f"rot_{i}"]) + jnp.sin(params[f"rot_{i}"])
        U.append(params[f"lin_w_{i}"][0] * c)
        V.append(params[f"lin_b_{i}"] * c)
        S.append(params[f"bn_g_{i}"] * inv_bn)
        T.append(params[f"bn_b_{i}"])
    U, V, S, T = (jnp.stack(a) for a in (U, V, S, T))

    w1 = params["gat1_w"]
    a1s = jnp.stack([w1[:, h * HIDDEN:(h + 1) * HIDDEN]
                     @ params["gat1_att_src"][h] for h in range(HEADS)],
                    axis=1)
    a1d = jnp.stack([w1[:, h * HIDDEN:(h + 1) * HIDDEN]
                     @ params["gat1_att_dst"][h] for h in range(HEADS)],
                    axis=1)
    b1 = params["gat1_bias"].reshape(1, HEADS * HIDDEN)
    w2 = params["gat2_w"]
    a2s = (w2 @ params["gat2_att_src"][0]).reshape(-1, 1)
    a2d = (w2 @ params["gat2_att_dst"][0]).reshape(-1, 1)
    b2 = params["gat2_bias"].reshape(1, HIDDEN)
    cw1 = params["cls_w1"]
    cb1 = params["cls_b1"].reshape(1, HIDDEN)
    cw2 = params["cls_w2"]
    cb2 = params["cls_b2"].reshape(1, 1)

    if B % 128 == 0:
        shp = (B // 128, 128)
    else:
        shp = (B, 1)
    stats = pl.pallas_call(
        _stats_kernel,
        out_shape=jax.ShapeDtypeStruct((1, 9), f32),
    )(omics_0.reshape(shp), omics_1.reshape(shp), omics_2.reshape(shp))

    s = stats[0]
    mu = s[0:3] / B
    pm = jnp.stack([jnp.stack([s[3], s[6], s[7]]),
                    jnp.stack([s[6], s[4], s[8]]),
                    jnp.stack([s[7], s[8], s[5]])])
    cov = pm - B * (mu[:, None] * mu[None, :])
    d = jnp.sqrt(jnp.diagonal(cov))
    corr = cov / (d[:, None] * d[None, :])
    allowed = (corr > 0.3) | jnp.eye(3, dtype=bool)
    abias = jnp.where(allowed, 0.0, NEG).astype(f32)

    G = 1024
    while B % G != 0:
        G //= 2
    nb = B // G
    row_spec = lambda shape: pl.BlockSpec(shape, lambda i: (0, 0))
    in_specs = [
        pl.BlockSpec((G, 1), lambda i: (i, 0)),
        pl.BlockSpec((G, 1), lambda i: (i, 0)),
        pl.BlockSpec((G, 1), lambda i: (i, 0)),
        row_spec((3, 3)),
        row_spec((3, HIDDEN)), row_spec((3, HIDDEN)),
        row_spec((3, HIDDEN)), row_spec((3, HIDDEN)),
        row_spec((HIDDEN, HEADS * HIDDEN)),
        row_spec((HIDDEN, HEADS)), row_spec((HIDDEN, HEADS)),
        row_spec((1, HEADS * HIDDEN)),
        row_spec((HEADS * HIDDEN, HIDDEN)),
        row_spec((HEADS * HIDDEN, 1)), row_spec((HEADS * HIDDEN, 1)),
        row_spec((1, HIDDEN)),
        row_spec((HIDDEN, HIDDEN)), row_spec((1, HIDDEN)),
        row_spec((HIDDEN, 1)), row_spec((1, 1)),
    ]
    out, pooled = pl.pallas_call(
        _fwd_kernel,
        grid=(nb,),
        in_specs=in_specs,
        out_specs=[pl.BlockSpec((G, 1), lambda i: (i, 0)),
                   pl.BlockSpec((G, HIDDEN), lambda i: (i, 0))],
        out_shape=[jax.ShapeDtypeStruct((B, 1), f32),
                   jax.ShapeDtypeStruct((B, HIDDEN), f32)],
    )(omics_0, omics_1, omics_2, abias, U, V, S, T,
      w1, a1s, a1d, b1, w2, a2s, a2d, b2, cw1, cb1, cw2, cb2)
    return out, pooled


# speedup vs baseline: 106.6678x; 1.0281x over previous
import math

import jax
import jax.numpy as jnp
from jax.experimental import pallas as pl

HIDDEN = 64
HEADS = 4
NEG = -1e30


def _stats_kernel(x0_ref, x1_ref, x2_ref, o_ref):
    a, b, c = x0_ref[...], x1_ref[...], x2_ref[...]
    vals = (a, b, c, a * a, b * b, c * c, a * b, a * c, b * c)
    o_ref[...] = jnp.concatenate(
        [jnp.sum(v, keepdims=True) for v in vals], axis=1)


def _lrelu(x):
    return jnp.where(x >= 0, x, 0.2 * x)


def _fwd_kernel(x0_ref, x1_ref, x2_ref, ab_ref,
                u_ref, v_ref, s_ref, t_ref,
                w1_ref, a1s_ref, a1d_ref, b1_ref,
                w2_ref, a2s_ref, a2d_ref, b2_ref,
                cw1_ref, cb1_ref, cw2_ref, cb2_ref,
                out_ref, pooled_ref):
    xs = (x0_ref[...], x1_ref[...], x2_ref[...])

    n = []
    for i in range(3):
        h = xs[i] * u_ref[i:i + 1, :] + v_ref[i:i + 1, :]
        h = jnp.maximum(h, 0.0)
        n.append(h * s_ref[i:i + 1, :] + t_ref[i:i + 1, :])

    w1 = w1_ref[...]
    h1 = [jnp.dot(ni, w1, preferred_element_type=jnp.float32) for ni in n]
    as1 = [jnp.dot(ni, a1s_ref[...], preferred_element_type=jnp.float32)
           for ni in n]
    ad1 = [jnp.dot(ni, a1d_ref[...], preferred_element_type=jnp.float32)
           for ni in n]
    x1 = []
    for j in range(3):
        al = [_lrelu(as1[i] + ad1[j]) + ab_ref[i:i + 1, j:j + 1]
              for i in range(3)]
        m = jnp.maximum(jnp.maximum(al[0], al[1]), al[2])
        e = [jnp.exp(a - m) for a in al]
        inv = 1.0 / (e[0] + e[1] + e[2] + 1e-16)
        att = [ei * inv for ei in e]
        parts = []
        for hd in range(4):
            lo = hd * HIDDEN
            acc = att[0][:, hd:hd + 1] * h1[0][:, lo:lo + HIDDEN]
            acc = acc + att[1][:, hd:hd + 1] * h1[1][:, lo:lo + HIDDEN]
            acc = acc + att[2][:, hd:hd + 1] * h1[2][:, lo:lo + HIDDEN]
            parts.append(acc)
        o = jnp.concatenate(parts, axis=1) + b1_ref[...]
        x1.append(jnp.where(o > 0, o, jnp.exp(jnp.minimum(o, 0.0)) - 1.0))

    w2 = w2_ref[...]
    h2 = [jnp.dot(xj, w2, preferred_element_type=jnp.float32) for xj in x1]
    as2 = [jnp.dot(xj, a2s_ref[...], preferred_element_type=jnp.float32)
           for xj in x1]
    ad2 = [jnp.dot(xj, a2d_ref[...], preferred_element_type=jnp.float32)
           for xj in x1]
    pooled = jnp.zeros_like(h2[0])
    for j in range(3):
        al = [_lrelu(as2[i] + ad2[j]) + ab_ref[i:i + 1, j:j + 1]
              for i in range(3)]
        m = jnp.maximum(jnp.maximum(al[0], al[1]), al[2])
        e = [jnp.exp(a - m) for a in al]
        inv = 1.0 / (e[0] + e[1] + e[2] + 1e-16)
        o = (e[0] * h2[0] + e[1] * h2[1] + e[2] * h2[2]) * inv
        pooled = pooled + (o + b2_ref[...])
    pooled = pooled * (1.0 / 3.0)
    pooled_ref[...] = pooled

    hc = jnp.dot(pooled, cw1_ref[...],
                 preferred_element_type=jnp.float32) + cb1_ref[...]
    hc = jnp.maximum(hc, 0.0)
    logit = jnp.dot(hc, cw2_ref[...],
                    preferred_element_type=jnp.float32) + cb2_ref[...]
    out_ref[...] = jax.nn.sigmoid(logit)


def kernel(omics_0, omics_1, omics_2, batch_size, params):
    B = omics_0.shape[0]
    f32 = jnp.float32

    U, V, S, T = [], [], [], []
    inv_bn = 1.0 / math.sqrt(1.0 + 1e-5)
    for i in range(3):
        c = jnp.cos(params[f"rot_{i}"]) + jnp.sin(params[f"rot_{i}"])
        U.append(params[f"lin_w_{i}"][0] * c)
        V.append(params[f"lin_b_{i}"] * c)
        S.append(params[f"bn_g_{i}"] * inv_bn)
        T.append(params[f"bn_b_{i}"])
    U, V, S, T = (jnp.stack(a) for a in (U, V, S, T))

    w1 = params["gat1_w"]
    a1s = jnp.stack([w1[:, h * HIDDEN:(h + 1) * HIDDEN]
                     @ params["gat1_att_src"][h] for h in range(HEADS)],
                    axis=1)
    a1d = jnp.stack([w1[:, h * HIDDEN:(h + 1) * HIDDEN]
                     @ params["gat1_att_dst"][h] for h in range(HEADS)],
                    axis=1)
    b1 = params["gat1_bias"].reshape(1, HEADS * HIDDEN)
    w2 = params["gat2_w"]
    a2s = (w2 @ params["gat2_att_src"][0]).reshape(-1, 1)
    a2d = (w2 @ params["gat2_att_dst"][0]).reshape(-1, 1)
    b2 = params["gat2_bias"].reshape(1, HIDDEN)
    cw1 = params["cls_w1"]
    cb1 = params["cls_b1"].reshape(1, HIDDEN)
    cw2 = params["cls_w2"]
    cb2 = params["cls_b2"].reshape(1, 1)

    if B % 128 == 0:
        shp = (B // 128, 128)
    else:
        shp = (B, 1)
    stats = pl.pallas_call(
        _stats_kernel,
        out_shape=jax.ShapeDtypeStruct((1, 9), f32),
    )(omics_0.reshape(shp), omics_1.reshape(shp), omics_2.reshape(shp))

    s = stats[0]
    mu = s[0:3] / B
    pm = jnp.stack([jnp.stack([s[3], s[6], s[7]]),
                    jnp.stack([s[6], s[4], s[8]]),
                    jnp.stack([s[7], s[8], s[5]])])
    cov = pm - B * (mu[:, None] * mu[None, :])
    d = jnp.sqrt(jnp.diagonal(cov))
    corr = cov / (d[:, None] * d[None, :])
    allowed = (corr > 0.3) | jnp.eye(3, dtype=bool)
    abias = jnp.where(allowed, 0.0, NEG).astype(f32)

    G = 2048
    while B % G != 0:
        G //= 2
    nb = B // G
    row_spec = lambda shape: pl.BlockSpec(shape, lambda i: (0, 0))
    in_specs = [
        pl.BlockSpec((G, 1), lambda i: (i, 0)),
        pl.BlockSpec((G, 1), lambda i: (i, 0)),
        pl.BlockSpec((G, 1), lambda i: (i, 0)),
        row_spec((3, 3)),
        row_spec((3, HIDDEN)), row_spec((3, HIDDEN)),
        row_spec((3, HIDDEN)), row_spec((3, HIDDEN)),
        row_spec((HIDDEN, HEADS * HIDDEN)),
        row_spec((HIDDEN, HEADS)), row_spec((HIDDEN, HEADS)),
        row_spec((1, HEADS * HIDDEN)),
        row_spec((HEADS * HIDDEN, HIDDEN)),
        row_spec((HEADS * HIDDEN, 1)), row_spec((HEADS * HIDDEN, 1)),
        row_spec((1, HIDDEN)),
        row_spec((HIDDEN, HIDDEN)), row_spec((1, HIDDEN)),
        row_spec((HIDDEN, 1)), row_spec((1, 1)),
    ]
    out, pooled = pl.pallas_call(
        _fwd_kernel,
        grid=(nb,),
        in_specs=in_specs,
        out_specs=[pl.BlockSpec((G, 1), lambda i: (i, 0)),
                   pl.BlockSpec((G, HIDDEN), lambda i: (i, 0))],
        out_shape=[jax.ShapeDtypeStruct((B, 1), f32),
                   jax.ShapeDtypeStruct((B, HIDDEN), f32)],
    )(omics_0, omics_1, omics_2, abias, U, V, S, T,
      w1, a1s, a1d, b1, w2, a2s, a2d, b2, cw1, cb1, cw2, cb2)
    return out, pooled


# speedup vs baseline: 247.8627x; 2.3237x over previous
import math

import jax
import jax.numpy as jnp
from jax.experimental import pallas as pl

HIDDEN = 64
HEADS = 4
NEG = -1e30


def _stats_kernel(x0_ref, x1_ref, x2_ref, o_ref):
    a, b, c = x0_ref[...], x1_ref[...], x2_ref[...]
    vals = (a, b, c, a * a, b * b, c * c, a * b, a * c, b * c)
    o_ref[...] = jnp.concatenate(
        [jnp.sum(v, keepdims=True) for v in vals], axis=1)


def _lrelu(x):
    return jnp.where(x >= 0, x, 0.2 * x)


def _fwd_kernel(x_ref, ab_ref,
                u_ref, v_ref, s_ref, t_ref,
                w1t_ref, a1s_ref, a1d_ref, b1_ref,
                w2t_ref, a2s_ref, a2d_ref, b2_ref,
                cw1t_ref, cb1_ref, cw2_ref, cb2_ref,
                out_ref, pooled_ref):
    H = HIDDEN

    n = []
    for i in range(3):
        xr = x_ref[i:i + 1, :]
        h = u_ref[i * H:(i + 1) * H, :] * xr + v_ref[i * H:(i + 1) * H, :]
        h = jnp.maximum(h, 0.0)
        n.append(h * s_ref[i * H:(i + 1) * H, :] + t_ref[i * H:(i + 1) * H, :])

    w1t = w1t_ref[...]
    h1 = [jnp.dot(w1t, ni, preferred_element_type=jnp.float32) for ni in n]
    as1 = [jnp.dot(a1s_ref[...], ni, preferred_element_type=jnp.float32)
           for ni in n]
    ad1 = [jnp.dot(a1d_ref[...], ni, preferred_element_type=jnp.float32)
           for ni in n]
    x1 = []
    for j in range(3):
        al = [_lrelu(as1[i] + ad1[j]) + ab_ref[i:i + 1, j:j + 1]
              for i in range(3)]
        m = jnp.maximum(jnp.maximum(al[0], al[1]), al[2])
        e = [jnp.exp(a - m) for a in al]
        inv = 1.0 / (e[0] + e[1] + e[2] + 1e-16)
        att = [ei * inv for ei in e]
        parts = []
        for hd in range(4):
            lo = hd * H
            acc = h1[0][lo:lo + H, :] * att[0][hd:hd + 1, :]
            acc = acc + h1[1][lo:lo + H, :] * att[1][hd:hd + 1, :]
            acc = acc + h1[2][lo:lo + H, :] * att[2][hd:hd + 1, :]
            parts.append(acc)
        o = jnp.concatenate(parts, axis=0) + b1_ref[...]
        x1.append(jnp.where(o > 0, o, jnp.exp(jnp.minimum(o, 0.0)) - 1.0))

    w2t = w2t_ref[...]
    h2 = [jnp.dot(w2t, xj, preferred_element_type=jnp.float32) for xj in x1]
    a2s = a2s_ref[...]
    a2d = a2d_ref[...]
    as2 = [jnp.sum(xj * a2s, axis=0, keepdims=True) for xj in x1]
    ad2 = [jnp.sum(xj * a2d, axis=0, keepdims=True) for xj in x1]
    pooled = jnp.zeros_like(h2[0])
    for j in range(3):
        al = [_lrelu(as2[i] + ad2[j]) + ab_ref[i:i + 1, j:j + 1]
              for i in range(3)]
        m = jnp.maximum(jnp.maximum(al[0], al[1]), al[2])
        e = [jnp.exp(a - m) for a in al]
        inv = 1.0 / (e[0] + e[1] + e[2] + 1e-16)
        o = (e[0] * h2[0] + e[1] * h2[1] + e[2] * h2[2]) * inv
        pooled = pooled + (o + b2_ref[...])
    pooled = pooled * (1.0 / 3.0)
    pooled_ref[...] = jnp.transpose(pooled)

    hc = jnp.dot(cw1t_ref[...], pooled,
                 preferred_element_type=jnp.float32) + cb1_ref[...]
    hc = jnp.maximum(hc, 0.0)
    logit = jnp.sum(hc * cw2_ref[...], axis=0, keepdims=True) + cb2_ref[...]
    out_ref[...] = jax.nn.sigmoid(logit)


def kernel(omics_0, omics_1, omics_2, batch_size, params):
    B = omics_0.shape[0]
    f32 = jnp.float32
    H = HIDDEN

    U, V, S, T = [], [], [], []
    inv_bn = 1.0 / math.sqrt(1.0 + 1e-5)
    for i in range(3):
        c = jnp.cos(params[f"rot_{i}"]) + jnp.sin(params[f"rot_{i}"])
        U.append(params[f"lin_w_{i}"][0] * c)
        V.append(params[f"lin_b_{i}"] * c)
        S.append(params[f"bn_g_{i}"] * inv_bn)
        T.append(params[f"bn_b_{i}"])
    U, V, S, T = (jnp.concatenate(a).reshape(3 * H, 1) for a in (U, V, S, T))

    w1 = params["gat1_w"]
    w1t = w1.T
    a1s = jnp.stack([w1[:, h * H:(h + 1) * H] @ params["gat1_att_src"][h]
                     for h in range(HEADS)])
    a1d = jnp.stack([w1[:, h * H:(h + 1) * H] @ params["gat1_att_dst"][h]
                     for h in range(HEADS)])
    b1 = params["gat1_bias"].reshape(HEADS * H, 1)
    w2t = params["gat2_w"].T
    a2s = (params["gat2_w"] @ params["gat2_att_src"][0]).reshape(-1, 1)
    a2d = (params["gat2_w"] @ params["gat2_att_dst"][0]).reshape(-1, 1)
    b2 = params["gat2_bias"].reshape(H, 1)
    cw1t = params["cls_w1"].T
    cb1 = params["cls_b1"].reshape(H, 1)
    cw2 = params["cls_w2"].reshape(H, 1)
    cb2 = params["cls_b2"].reshape(1, 1)

    if B % 128 == 0:
        shp = (B // 128, 128)
    else:
        shp = (B, 1)
    stats = pl.pallas_call(
        _stats_kernel,
        out_shape=jax.ShapeDtypeStruct((1, 9), f32),
    )(omics_0.reshape(shp), omics_1.reshape(shp), omics_2.reshape(shp))

    s = stats[0]
    mu = s[0:3] / B
    pm = jnp.stack([jnp.stack([s[3], s[6], s[7]]),
                    jnp.stack([s[6], s[4], s[8]]),
                    jnp.stack([s[7], s[8], s[5]])])
    cov = pm - B * (mu[:, None] * mu[None, :])
    d = jnp.sqrt(jnp.diagonal(cov))
    corr = cov / (d[:, None] * d[None, :])
    allowed = (corr > 0.3) | jnp.eye(3, dtype=bool)
    abias = jnp.where(allowed, 0.0, NEG).astype(f32)

    x = jnp.concatenate([omics_0.reshape(1, B), omics_1.reshape(1, B),
                         omics_2.reshape(1, B)], axis=0)

    G = 1024
    while B % G != 0:
        G //= 2
    nb = B // G
    full = lambda shape: pl.BlockSpec(shape, lambda i: (0, 0))
    in_specs = [
        pl.BlockSpec((3, G), lambda i: (0, i)),
        full((3, 3)),
        full((3 * H, 1)), full((3 * H, 1)), full((3 * H, 1)), full((3 * H, 1)),
        full((HEADS * H, H)),
        full((HEADS, H)), full((HEADS, H)),
        full((HEADS * H, 1)),
        full((H, HEADS * H)),
        full((HEADS * H, 1)), full((HEADS * H, 1)),
        full((H, 1)),
        full((H, H)), full((H, 1)),
        full((H, 1)), full((1, 1)),
    ]
    out, pooled = pl.pallas_call(
        _fwd_kernel,
        grid=(nb,),
        in_specs=in_specs,
        out_specs=[pl.BlockSpec((1, G), lambda i: (0, i)),
                   pl.BlockSpec((G, H), lambda i: (i, 0))],
        out_shape=[jax.ShapeDtypeStruct((1, B), f32),
                   jax.ShapeDtypeStruct((B, H), f32)],
    )(x, abias, U, V, S, T,
      w1t, a1s, a1d, b1, w2t, a2s, a2d, b2, cw1t, cb1, cw2, cb2)
    return out.reshape(B, 1), pooled


# speedup vs baseline: 276.1495x; 1.1141x over previous
import math

import jax
import jax.numpy as jnp
from jax.experimental import pallas as pl

HIDDEN = 64
HEADS = 4
NEG = -1e30


def _stats_kernel(x0_ref, x1_ref, x2_ref, o_ref):
    a, b, c = x0_ref[...], x1_ref[...], x2_ref[...]
    vals = (a, b, c, a * a, b * b, c * c, a * b, a * c, b * c)
    o_ref[...] = jnp.concatenate(
        [jnp.sum(v, keepdims=True) for v in vals], axis=1)


def _lrelu(x):
    return jnp.where(x >= 0, x, 0.2 * x)


def _fwd_kernel(x_ref, ab_ref,
                u_ref, v_ref, s_ref, t_ref,
                w1t_ref, a1s_ref, a1d_ref, b1_ref,
                w2t_ref, a2s_ref, a2d_ref, b2_ref,
                cw1t_ref, cb1_ref, cw2_ref, cb2_ref,
                out_ref, pooled_ref):
    H = HIDDEN

    n = []
    for i in range(3):
        xr = x_ref[i:i + 1, :]
        h = u_ref[i * H:(i + 1) * H, :] * xr + v_ref[i * H:(i + 1) * H, :]
        h = jnp.maximum(h, 0.0)
        n.append(h * s_ref[i * H:(i + 1) * H, :] + t_ref[i * H:(i + 1) * H, :])

    w1t = w1t_ref[...]
    h1 = [jnp.dot(w1t, ni, preferred_element_type=jnp.float32) for ni in n]
    as1 = [jnp.dot(a1s_ref[...], ni, preferred_element_type=jnp.float32)
           for ni in n]
    ad1 = [jnp.dot(a1d_ref[...], ni, preferred_element_type=jnp.float32)
           for ni in n]
    x1 = []
    for j in range(3):
        al = [_lrelu(as1[i] + ad1[j]) + ab_ref[i:i + 1, j:j + 1]
              for i in range(3)]
        m = jnp.maximum(jnp.maximum(al[0], al[1]), al[2])
        e = [jnp.exp(a - m) for a in al]
        inv = 1.0 / (e[0] + e[1] + e[2] + 1e-16)
        att = [ei * inv for ei in e]
        parts = []
        for hd in range(4):
            lo = hd * H
            acc = h1[0][lo:lo + H, :] * att[0][hd:hd + 1, :]
            acc = acc + h1[1][lo:lo + H, :] * att[1][hd:hd + 1, :]
            acc = acc + h1[2][lo:lo + H, :] * att[2][hd:hd + 1, :]
            parts.append(acc)
        o = jnp.concatenate(parts, axis=0) + b1_ref[...]
        x1.append(jnp.where(o > 0, o, jnp.exp(jnp.minimum(o, 0.0)) - 1.0))

    w2t = w2t_ref[...]
    h2 = [jnp.dot(w2t, xj, preferred_element_type=jnp.float32) for xj in x1]
    a2s = a2s_ref[...]
    a2d = a2d_ref[...]
    as2 = [jnp.sum(xj * a2s, axis=0, keepdims=True) for xj in x1]
    ad2 = [jnp.sum(xj * a2d, axis=0, keepdims=True) for xj in x1]
    pooled = jnp.zeros_like(h2[0])
    for j in range(3):
        al = [_lrelu(as2[i] + ad2[j]) + ab_ref[i:i + 1, j:j + 1]
              for i in range(3)]
        m = jnp.maximum(jnp.maximum(al[0], al[1]), al[2])
        e = [jnp.exp(a - m) for a in al]
        inv = 1.0 / (e[0] + e[1] + e[2] + 1e-16)
        o = (e[0] * h2[0] + e[1] * h2[1] + e[2] * h2[2]) * inv
        pooled = pooled + (o + b2_ref[...])
    pooled = pooled * (1.0 / 3.0)
    pooled_ref[...] = jnp.transpose(pooled)

    hc = jnp.dot(cw1t_ref[...], pooled,
                 preferred_element_type=jnp.float32) + cb1_ref[...]
    hc = jnp.maximum(hc, 0.0)
    logit = jnp.sum(hc * cw2_ref[...], axis=0, keepdims=True) + cb2_ref[...]
    out_ref[...] = jax.nn.sigmoid(logit)


def kernel(omics_0, omics_1, omics_2, batch_size, params):
    B = omics_0.shape[0]
    f32 = jnp.float32
    H = HIDDEN

    U, V, S, T = [], [], [], []
    inv_bn = 1.0 / math.sqrt(1.0 + 1e-5)
    for i in range(3):
        c = jnp.cos(params[f"rot_{i}"]) + jnp.sin(params[f"rot_{i}"])
        U.append(params[f"lin_w_{i}"][0] * c)
        V.append(params[f"lin_b_{i}"] * c)
        S.append(params[f"bn_g_{i}"] * inv_bn)
        T.append(params[f"bn_b_{i}"])
    U, V, S, T = (jnp.concatenate(a).reshape(3 * H, 1) for a in (U, V, S, T))

    w1 = params["gat1_w"]
    w1t = w1.T
    a1s = jnp.stack([w1[:, h * H:(h + 1) * H] @ params["gat1_att_src"][h]
                     for h in range(HEADS)])
    a1d = jnp.stack([w1[:, h * H:(h + 1) * H] @ params["gat1_att_dst"][h]
                     for h in range(HEADS)])
    b1 = params["gat1_bias"].reshape(HEADS * H, 1)
    w2t = params["gat2_w"].T
    a2s = (params["gat2_w"] @ params["gat2_att_src"][0]).reshape(-1, 1)
    a2d = (params["gat2_w"] @ params["gat2_att_dst"][0]).reshape(-1, 1)
    b2 = params["gat2_bias"].reshape(H, 1)
    cw1t = params["cls_w1"].T
    cb1 = params["cls_b1"].reshape(H, 1)
    cw2 = params["cls_w2"].reshape(H, 1)
    cb2 = params["cls_b2"].reshape(1, 1)

    if B % 128 == 0:
        shp = (B // 128, 128)
    else:
        shp = (B, 1)
    stats = pl.pallas_call(
        _stats_kernel,
        out_shape=jax.ShapeDtypeStruct((1, 9), f32),
    )(omics_0.reshape(shp), omics_1.reshape(shp), omics_2.reshape(shp))

    s = stats[0]
    mu = s[0:3] / B
    pm = jnp.stack([jnp.stack([s[3], s[6], s[7]]),
                    jnp.stack([s[6], s[4], s[8]]),
                    jnp.stack([s[7], s[8], s[5]])])
    cov = pm - B * (mu[:, None] * mu[None, :])
    d = jnp.sqrt(jnp.diagonal(cov))
    corr = cov / (d[:, None] * d[None, :])
    allowed = (corr > 0.3) | jnp.eye(3, dtype=bool)
    abias = jnp.where(allowed, 0.0, NEG).astype(f32)

    x = jnp.concatenate([omics_0.reshape(1, B), omics_1.reshape(1, B),
                         omics_2.reshape(1, B)], axis=0)

    G = 2048
    while B % G != 0:
        G //= 2
    nb = B // G
    full = lambda shape: pl.BlockSpec(shape, lambda i: (0, 0))
    in_specs = [
        pl.BlockSpec((3, G), lambda i: (0, i)),
        full((3, 3)),
        full((3 * H, 1)), full((3 * H, 1)), full((3 * H, 1)), full((3 * H, 1)),
        full((HEADS * H, H)),
        full((HEADS, H)), full((HEADS, H)),
        full((HEADS * H, 1)),
        full((H, HEADS * H)),
        full((HEADS * H, 1)), full((HEADS * H, 1)),
        full((H, 1)),
        full((H, H)), full((H, 1)),
        full((H, 1)), full((1, 1)),
    ]
    out, pooled = pl.pallas_call(
        _fwd_kernel,
        grid=(nb,),
        in_specs=in_specs,
        out_specs=[pl.BlockSpec((1, G), lambda i: (0, i)),
                   pl.BlockSpec((G, H), lambda i: (i, 0))],
        out_shape=[jax.ShapeDtypeStruct((1, B), f32),
                   jax.ShapeDtypeStruct((B, H), f32)],
    )(x, abias, U, V, S, T,
      w1t, a1s, a1d, b1, w2t, a2s, a2d, b2, cw1t, cb1, cw2, cb2)
    return out.reshape(B, 1), pooled


# speedup vs baseline: 283.1232x; 1.0253x over previous
import math

import jax
import jax.numpy as jnp
from jax.experimental import pallas as pl

HIDDEN = 64
HEADS = 4
NEG = -1e30


def _stats_kernel(x0_ref, x1_ref, x2_ref, o_ref):
    a, b, c = x0_ref[...], x1_ref[...], x2_ref[...]
    vals = (a, b, c, a * a, b * b, c * c, a * b, a * c, b * c)
    o_ref[...] = jnp.concatenate(
        [jnp.sum(v, keepdims=True) for v in vals], axis=1)


def _lrelu(x):
    return jnp.where(x >= 0, x, 0.2 * x)


def _fwd_kernel(x_ref, ab_ref,
                u_ref, v_ref, s_ref, t_ref,
                w1t_ref, a1s_ref, a1d_ref, b1_ref,
                w2t_ref, a2s_ref, a2d_ref, b2_ref,
                cw1t_ref, cb1_ref, cw2_ref, cb2_ref,
                out_ref, pooled_ref):
    H = HIDDEN

    n = []
    for i in range(3):
        xr = x_ref[i:i + 1, :]
        h = u_ref[i * H:(i + 1) * H, :] * xr + v_ref[i * H:(i + 1) * H, :]
        h = jnp.maximum(h, 0.0)
        n.append(h * s_ref[i * H:(i + 1) * H, :] + t_ref[i * H:(i + 1) * H, :])

    w1t = w1t_ref[...]
    h1 = [jnp.dot(w1t, ni, preferred_element_type=jnp.float32) for ni in n]
    as1 = [jnp.dot(a1s_ref[...], ni, preferred_element_type=jnp.float32)
           for ni in n]
    ad1 = [jnp.dot(a1d_ref[...], ni, preferred_element_type=jnp.float32)
           for ni in n]
    x1 = []
    for j in range(3):
        al = [_lrelu(as1[i] + ad1[j]) + ab_ref[i:i + 1, j:j + 1]
              for i in range(3)]
        m = jnp.maximum(jnp.maximum(al[0], al[1]), al[2])
        e = [jnp.exp(a - m) for a in al]
        inv = 1.0 / (e[0] + e[1] + e[2] + 1e-16)
        att = [ei * inv for ei in e]
        parts = []
        for hd in range(4):
            lo = hd * H
            acc = h1[0][lo:lo + H, :] * att[0][hd:hd + 1, :]
            acc = acc + h1[1][lo:lo + H, :] * att[1][hd:hd + 1, :]
            acc = acc + h1[2][lo:lo + H, :] * att[2][hd:hd + 1, :]
            parts.append(acc)
        o = jnp.concatenate(parts, axis=0) + b1_ref[...]
        x1.append(jnp.where(o > 0, o, jnp.exp(jnp.minimum(o, 0.0)) - 1.0))

    w2t = w2t_ref[...]
    h2 = [jnp.dot(w2t, xj, preferred_element_type=jnp.float32) for xj in x1]
    a2s = a2s_ref[...]
    a2d = a2d_ref[...]
    as2 = [jnp.sum(xj * a2s, axis=0, keepdims=True) for xj in x1]
    ad2 = [jnp.sum(xj * a2d, axis=0, keepdims=True) for xj in x1]
    pooled = jnp.zeros_like(h2[0])
    for j in range(3):
        al = [_lrelu(as2[i] + ad2[j]) + ab_ref[i:i + 1, j:j + 1]
              for i in range(3)]
        m = jnp.maximum(jnp.maximum(al[0], al[1]), al[2])
        e = [jnp.exp(a - m) for a in al]
        inv = 1.0 / (e[0] + e[1] + e[2] + 1e-16)
        o = (e[0] * h2[0] + e[1] * h2[1] + e[2] * h2[2]) * inv
        pooled = pooled + (o + b2_ref[...])
    pooled = pooled * (1.0 / 3.0)
    pooled_ref[...] = jnp.transpose(pooled)

    hc = jnp.dot(cw1t_ref[...], pooled,
                 preferred_element_type=jnp.float32) + cb1_ref[...]
    hc = jnp.maximum(hc, 0.0)
    logit = jnp.sum(hc * cw2_ref[...], axis=0, keepdims=True) + cb2_ref[...]
    out_ref[...] = jax.nn.sigmoid(logit)


def kernel(omics_0, omics_1, omics_2, batch_size, params):
    B = omics_0.shape[0]
    f32 = jnp.float32
    H = HIDDEN

    U, V, S, T = [], [], [], []
    inv_bn = 1.0 / math.sqrt(1.0 + 1e-5)
    for i in range(3):
        c = jnp.cos(params[f"rot_{i}"]) + jnp.sin(params[f"rot_{i}"])
        U.append(params[f"lin_w_{i}"][0] * c)
        V.append(params[f"lin_b_{i}"] * c)
        S.append(params[f"bn_g_{i}"] * inv_bn)
        T.append(params[f"bn_b_{i}"])
    U, V, S, T = (jnp.concatenate(a).reshape(3 * H, 1) for a in (U, V, S, T))

    w1 = params["gat1_w"]
    w1t = w1.T
    a1s = jnp.stack([w1[:, h * H:(h + 1) * H] @ params["gat1_att_src"][h]
                     for h in range(HEADS)])
    a1d = jnp.stack([w1[:, h * H:(h + 1) * H] @ params["gat1_att_dst"][h]
                     for h in range(HEADS)])
    b1 = params["gat1_bias"].reshape(HEADS * H, 1)
    w2t = params["gat2_w"].T
    a2s = (params["gat2_w"] @ params["gat2_att_src"][0]).reshape(-1, 1)
    a2d = (params["gat2_w"] @ params["gat2_att_dst"][0]).reshape(-1, 1)
    b2 = params["gat2_bias"].reshape(H, 1)
    cw1t = params["cls_w1"].T
    cb1 = params["cls_b1"].reshape(H, 1)
    cw2 = params["cls_w2"].reshape(H, 1)
    cb2 = params["cls_b2"].reshape(1, 1)

    if B % 128 == 0:
        shp = (B // 128, 128)
    else:
        shp = (B, 1)
    stats = pl.pallas_call(
        _stats_kernel,
        out_shape=jax.ShapeDtypeStruct((1, 9), f32),
    )(omics_0.reshape(shp), omics_1.reshape(shp), omics_2.reshape(shp))

    s = stats[0]
    mu = s[0:3] / B
    pm = jnp.stack([jnp.stack([s[3], s[6], s[7]]),
                    jnp.stack([s[6], s[4], s[8]]),
                    jnp.stack([s[7], s[8], s[5]])])
    cov = pm - B * (mu[:, None] * mu[None, :])
    d = jnp.sqrt(jnp.diagonal(cov))
    corr = cov / (d[:, None] * d[None, :])
    allowed = (corr > 0.3) | jnp.eye(3, dtype=bool)
    abias = jnp.where(allowed, 0.0, NEG).astype(f32)

    x = jnp.concatenate([omics_0.reshape(1, B), omics_1.reshape(1, B),
                         omics_2.reshape(1, B)], axis=0)

    G = 4096
    while B % G != 0:
        G //= 2
    nb = B // G
    full = lambda shape: pl.BlockSpec(shape, lambda i: (0, 0))
    in_specs = [
        pl.BlockSpec((3, G), lambda i: (0, i)),
        full((3, 3)),
        full((3 * H, 1)), full((3 * H, 1)), full((3 * H, 1)), full((3 * H, 1)),
        full((HEADS * H, H)),
        full((HEADS, H)), full((HEADS, H)),
        full((HEADS * H, 1)),
        full((H, HEADS * H)),
        full((HEADS * H, 1)), full((HEADS * H, 1)),
        full((H, 1)),
        full((H, H)), full((H, 1)),
        full((H, 1)), full((1, 1)),
    ]
    out, pooled = pl.pallas_call(
        _fwd_kernel,
        grid=(nb,),
        in_specs=in_specs,
        out_specs=[pl.BlockSpec((1, G), lambda i: (0, i)),
                   pl.BlockSpec((G, H), lambda i: (i, 0))],
        out_shape=[jax.ShapeDtypeStruct((1, B), f32),
                   jax.ShapeDtypeStruct((B, H), f32)],
    )(x, abias, U, V, S, T,
      w1t, a1s, a1d, b1, w2t, a2s, a2d, b2, cw1t, cb1, cw2, cb2)
    return out.reshape(B, 1), pooled


# speedup vs baseline: 307.5324x; 1.0862x over previous
import math

import jax
import jax.numpy as jnp
from jax import lax
from jax.experimental import pallas as pl
from jax.experimental.pallas import tpu as pltpu

HIDDEN = 64
HEADS = 4
NEG = -1e30


def _lrelu(x):
    return jnp.where(x >= 0, x, 0.2 * x)


def _fwd_kernel(xfull_ref, x_ref,
                u_ref, v_ref, s_ref, t_ref,
                w1t_ref, a1s_ref, a1d_ref, b1_ref,
                w2t_ref, a2s_ref, a2d_ref, b2_ref,
                cw1t_ref, cb1_ref, cw2_ref, cb2_ref,
                out_ref, pooled_ref, ab_ref):
    H = HIDDEN
    pid = pl.program_id(0)

    @pl.when(pid == 0)
    def _stats():
        r = xfull_ref[...]
        B = r.shape[1]
        sums = jnp.sum(r, axis=1, keepdims=True)
        gram = lax.dot_general(r, r, (((1,), (1,)), ((), ())),
                               preferred_element_type=jnp.float32)
        mu = sums * (1.0 / B)
        mu_row = jnp.concatenate(
            [mu[i:i + 1, 0:1] for i in range(3)], axis=1)
        cov = gram - B * (mu * mu_row)
        dcol = jnp.concatenate(
            [cov[i:i + 1, i:i + 1] for i in range(3)], axis=0)
        drow = jnp.concatenate(
            [cov[i:i + 1, i:i + 1] for i in range(3)], axis=1)
        thr = 0.3 * jnp.sqrt(dcol * drow)
        rows = lax.broadcasted_iota(jnp.int32, (3, 3), 0)
        cols = lax.broadcasted_iota(jnp.int32, (3, 3), 1)
        allowed = (cov > thr) | (rows == cols)
        ab_ref[...] = jnp.where(allowed, 0.0, NEG)

    @pl.when(pid > 0)
    def _forward():
        n = []
        for i in range(3):
            xr = x_ref[i:i + 1, :]
            h = u_ref[i * H:(i + 1) * H, :] * xr + v_ref[i * H:(i + 1) * H, :]
            h = jnp.maximum(h, 0.0)
            n.append(h * s_ref[i * H:(i + 1) * H, :]
                     + t_ref[i * H:(i + 1) * H, :])

        w1t = w1t_ref[...]
        h1 = [jnp.dot(w1t, ni, preferred_element_type=jnp.float32)
              for ni in n]
        as1 = [jnp.dot(a1s_ref[...], ni, preferred_element_type=jnp.float32)
               for ni in n]
        ad1 = [jnp.dot(a1d_ref[...], ni, preferred_element_type=jnp.float32)
               for ni in n]
        x1 = []
        for j in range(3):
            al = [_lrelu(as1[i] + ad1[j]) + ab_ref[i:i + 1, j:j + 1]
                  for i in range(3)]
            m = jnp.maximum(jnp.maximum(al[0], al[1]), al[2])
            e = [jnp.exp(a - m) for a in al]
            inv = 1.0 / (e[0] + e[1] + e[2] + 1e-16)
            att = [ei * inv for ei in e]
            parts = []
            for hd in range(4):
                lo = hd * H
                acc = h1[0][lo:lo + H, :] * att[0][hd:hd + 1, :]
                acc = acc + h1[1][lo:lo + H, :] * att[1][hd:hd + 1, :]
                acc = acc + h1[2][lo:lo + H, :] * att[2][hd:hd + 1, :]
                parts.append(acc)
            o = jnp.concatenate(parts, axis=0) + b1_ref[...]
            x1.append(jnp.where(o > 0, o,
                                jnp.exp(jnp.minimum(o, 0.0)) - 1.0))

        w2t = w2t_ref[...]
        h2 = [jnp.dot(w2t, xj, preferred_element_type=jnp.float32)
              for xj in x1]
        a2s = a2s_ref[...]
        a2d = a2d_ref[...]
        as2 = [jnp.sum(xj * a2s, axis=0, keepdims=True) for xj in x1]
        ad2 = [jnp.sum(xj * a2d, axis=0, keepdims=True) for xj in x1]
        pooled = jnp.zeros_like(h2[0])
        for j in range(3):
            al = [_lrelu(as2[i] + ad2[j]) + ab_ref[i:i + 1, j:j + 1]
                  for i in range(3)]
            m = jnp.maximum(jnp.maximum(al[0], al[1]), al[2])
            e = [jnp.exp(a - m) for a in al]
            inv = 1.0 / (e[0] + e[1] + e[2] + 1e-16)
            pooled = pooled + (e[0] * h2[0] + e[1] * h2[1]
                               + e[2] * h2[2]) * inv
        pooled = pooled * (1.0 / 3.0) + b2_ref[...]
        pooled_ref[...] = jnp.transpose(pooled)

        hc = jnp.dot(cw1t_ref[...], pooled,
                     preferred_element_type=jnp.float32) + cb1_ref[...]
        hc = jnp.maximum(hc, 0.0)
        logit = (jnp.sum(hc * cw2_ref[...], axis=0, keepdims=True)
                 + cb2_ref[...])
        out_ref[...] = jax.nn.sigmoid(logit)


def kernel(omics_0, omics_1, omics_2, batch_size, params):
    B = omics_0.shape[0]
    f32 = jnp.float32
    H = HIDDEN

    U, V, S, T = [], [], [], []
    inv_bn = 1.0 / math.sqrt(1.0 + 1e-5)
    for i in range(3):
        c = jnp.cos(params[f"rot_{i}"]) + jnp.sin(params[f"rot_{i}"])
        U.append(params[f"lin_w_{i}"][0] * c)
        V.append(params[f"lin_b_{i}"] * c)
        S.append(params[f"bn_g_{i}"] * inv_bn)
        T.append(params[f"bn_b_{i}"])
    U, V, S, T = (jnp.concatenate(a).reshape(3 * H, 1) for a in (U, V, S, T))

    w1 = params["gat1_w"]
    w1t = w1.T
    a1s = jnp.stack([w1[:, h * H:(h + 1) * H] @ params["gat1_att_src"][h]
                     for h in range(HEADS)])
    a1d = jnp.stack([w1[:, h * H:(h + 1) * H] @ params["gat1_att_dst"][h]
                     for h in range(HEADS)])
    b1 = params["gat1_bias"].reshape(HEADS * H, 1)
    w2t = params["gat2_w"].T
    a2s = (params["gat2_w"] @ params["gat2_att_src"][0]).reshape(-1, 1)
    a2d = (params["gat2_w"] @ params["gat2_att_dst"][0]).reshape(-1, 1)
    b2 = params["gat2_bias"].reshape(H, 1)
    cw1t = params["cls_w1"].T
    cb1 = params["cls_b1"].reshape(H, 1)
    cw2 = params["cls_w2"].reshape(H, 1)
    cb2 = params["cls_b2"].reshape(1, 1)

    x = jnp.concatenate([omics_0.reshape(1, B), omics_1.reshape(1, B),
                         omics_2.reshape(1, B)], axis=0)

    G = 4096
    while B % G != 0:
        G //= 2
    nb = B // G
    full = lambda shape: pl.BlockSpec(shape, lambda i: (0, 0))
    blk = lambda i: (0, jnp.maximum(i - 1, 0))
    in_specs = [
        full((3, B)),
        pl.BlockSpec((3, G), blk),
        full((3 * H, 1)), full((3 * H, 1)), full((3 * H, 1)), full((3 * H, 1)),
        full((HEADS * H, H)),
        full((HEADS, H)), full((HEADS, H)),
        full((HEADS * H, 1)),
        full((H, HEADS * H)),
        full((HEADS * H, 1)), full((HEADS * H, 1)),
        full((H, 1)),
        full((H, H)), full((H, 1)),
        full((H, 1)), full((1, 1)),
    ]
    out, pooled = pl.pallas_call(
        _fwd_kernel,
        grid=(nb + 1,),
        in_specs=in_specs,
        out_specs=[
            pl.BlockSpec((1, G), lambda i: (0, jnp.maximum(i - 1, 0))),
            pl.BlockSpec((G, H), lambda i: (jnp.maximum(i - 1, 0), 0)),
        ],
        out_shape=[jax.ShapeDtypeStruct((1, B), f32),
                   jax.ShapeDtypeStruct((B, H), f32)],
        scratch_shapes=[pltpu.VMEM((3, 3), f32)],
    )(x, x, U, V, S, T,
      w1t, a1s, a1d, b1, w2t, a2s, a2d, b2, cw1t, cb1, cw2, cb2)
    return out.reshape(B, 1), pooled
